# baseline (speedup 1.0000x reference)
import functools

import jax
import jax.numpy as jnp
from jax import lax
from jax.experimental import pallas as pl
from jax.experimental.pallas import tpu as pltpu
from jax.experimental.pallas import tpu_sc as plsc


def _tc_add_kernel(x_ref, pos_ref, out_ref):
    out_ref[...] = x_ref[...] + pos_ref[...][None]


def _tc_kernel(x, pos_table):
    batch, seq, dim = x.shape
    s_blk = 2048
    n_seq = seq // s_blk
    return pl.pallas_call(
        _tc_add_kernel,
        grid=(n_seq, batch),
        in_specs=[
            pl.BlockSpec((1, s_blk, dim), lambda i, j: (j, i, 0)),
            pl.BlockSpec((s_blk, dim), lambda i, j: (i, 0)),
        ],
        out_specs=pl.BlockSpec((1, s_blk, dim), lambda i, j: (j, i, 0)),
        out_shape=jax.ShapeDtypeStruct(x.shape, x.dtype),
    )(x, pos_table[:seq])


_LANES = 16
_ROWS_PER_CHUNK = 16
_NBUF = 4


def _make_sc_kernel(batch, seq, dim):
    info = plsc.get_sparse_core_info()
    nc, ns = info.num_cores, info.num_subcores
    nw = nc * ns
    s_per_w = seq // nw
    n_chunks = s_per_w // _ROWS_PER_CHUNK
    chunk = _ROWS_PER_CHUNK * dim
    steps = n_chunks * batch

    mesh = plsc.VectorSubcoreMesh(core_axis_name="c", subcore_axis_name="s")

    scratch = (
        [pltpu.VMEM((chunk,), jnp.float32) for _ in range(_NBUF + 2)]
        + [pltpu.SemaphoreType.DMA for _ in range(2 * _NBUF + 2)]
    )

    @functools.partial(
        pl.kernel,
        mesh=mesh,
        out_type=jax.ShapeDtypeStruct((batch * seq * dim,), jnp.float32),
        scratch_types=scratch,
    )
    def sc_k(x_hbm, pos_hbm, out_hbm, *refs):
        xbufs = refs[:_NBUF]
        pbufs = refs[_NBUF:_NBUF + 2]
        xsems = refs[_NBUF + 2:2 * _NBUF + 2]
        osems = refs[2 * _NBUF + 2:3 * _NBUF + 2]
        psems = refs[3 * _NBUF + 2:]

        wid = lax.axis_index("s") * nc + lax.axis_index("c")
        s_base = wid * s_per_w

        def pos_off(c):
            return (s_base + c * _ROWS_PER_CHUNK) * dim

        def x_off(t):
            c, b = divmod(t, batch)
            return b * seq * dim + pos_off(c)

        def start_x(t):
            i = t % _NBUF
            return pltpu.async_copy(
                x_hbm.at[pl.ds(x_off(t), chunk)], xbufs[i], xsems[i])

        def start_pos(c):
            j = c % 2
            return pltpu.async_copy(
                pos_hbm.at[pl.ds(pos_off(c), chunk)], pbufs[j], psems[j])

        hx, hout, hpos = {}, {}, {}
        hpos[0] = start_pos(0)
        for t in range(min(_NBUF - 1, steps)):
            hx[t] = start_x(t)

        for t in range(steps):
            c, b = divmod(t, batch)
            i = t % _NBUF
            if b == 0:
                hpos[c].wait()
                if c + 1 < n_chunks:
                    hpos[c + 1] = start_pos(c + 1)
            hx[t].wait()

            xb, pb = xbufs[i], pbufs[c % 2]

            @plsc.parallel_loop(0, chunk // _LANES, unroll=8)
            def _add(j):
                sl = pl.ds(j * _LANES, _LANES)
                plsc.addupdate(xb.at[sl], pb[sl])
            hout[t] = pltpu.async_copy(
                xb, out_hbm.at[pl.ds(x_off(t), chunk)], osems[i])

            p = t + _NBUF - 1
            if p < steps:
                q = p - _NBUF
                if q >= 0:
                    hout[q].wait()
                hx[p] = start_x(p)

        for t in range(max(0, steps - _NBUF), steps):
            hout[t].wait()

    return sc_k


def _sc_kernel(x, pos_table):
    batch, seq, dim = x.shape
    sc_k = _make_sc_kernel(batch, seq, dim)
    out = sc_k(x.reshape(-1), pos_table[:seq].reshape(-1))
    return out.reshape(batch, seq, dim)


def kernel(x, pos_table):
    return _sc_kernel(x, pos_table)


# speedup vs baseline: 4.1993x; 4.1993x over previous
import functools

import jax
import jax.numpy as jnp
from jax import lax
from jax.experimental import pallas as pl
from jax.experimental.pallas import tpu as pltpu
from jax.experimental.pallas import tpu_sc as plsc


def _tc_add_kernel(x_ref, pos_ref, out_ref):
    out_ref[...] = x_ref[...] + pos_ref[...][None]


def _tc_kernel(x, pos_table):
    batch, seq, dim = x.shape
    s_blk = 2048
    n_seq = seq // s_blk
    return pl.pallas_call(
        _tc_add_kernel,
        grid=(n_seq, batch),
        in_specs=[
            pl.BlockSpec((1, s_blk, dim), lambda i, j: (j, i, 0)),
            pl.BlockSpec((s_blk, dim), lambda i, j: (i, 0)),
        ],
        out_specs=pl.BlockSpec((1, s_blk, dim), lambda i, j: (j, i, 0)),
        out_shape=jax.ShapeDtypeStruct(x.shape, x.dtype),
    )(x, pos_table[:seq])


_LANES = 16
_ROWS_PER_CHUNK = 16
_NBUF = 4


def _make_sc_kernel(batch, seq, dim):
    info = plsc.get_sparse_core_info()
    nc, ns = info.num_cores, info.num_subcores
    nw = nc * ns
    s_per_w = seq // nw
    n_chunks = s_per_w // _ROWS_PER_CHUNK
    chunk = _ROWS_PER_CHUNK * dim
    steps = n_chunks * batch

    mesh = plsc.VectorSubcoreMesh(core_axis_name="c", subcore_axis_name="s")

    scratch = (
        [pltpu.VMEM((chunk,), jnp.float32) for _ in range(_NBUF + 2)]
        + [pltpu.SemaphoreType.DMA for _ in range(2 * _NBUF + 2)]
    )

    @functools.partial(
        pl.kernel,
        mesh=mesh,
        out_type=jax.ShapeDtypeStruct((batch * seq * dim,), jnp.float32),
        scratch_types=scratch,
    )
    def sc_k(x_hbm, pos_hbm, out_hbm, *refs):
        xbufs = refs[:_NBUF]
        pbufs = refs[_NBUF:_NBUF + 2]
        xsems = refs[_NBUF + 2:2 * _NBUF + 2]
        osems = refs[2 * _NBUF + 2:3 * _NBUF + 2]
        psems = refs[3 * _NBUF + 2:]

        wid = lax.axis_index("s") * nc + lax.axis_index("c")
        s_base = wid * s_per_w

        def pos_off(c):
            return (s_base + c * _ROWS_PER_CHUNK) * dim

        def x_off(t):
            c, b = divmod(t, batch)
            return b * seq * dim + pos_off(c)

        def start_x(t):
            i = t % _NBUF
            return pltpu.async_copy(
                x_hbm.at[pl.ds(x_off(t), chunk)], xbufs[i], xsems[i])

        def start_pos(c):
            j = c % 2
            return pltpu.async_copy(
                pos_hbm.at[pl.ds(pos_off(c), chunk)], pbufs[j], psems[j])

        hx, hout, hpos = {}, {}, {}
        hpos[0] = start_pos(0)
        for t in range(min(_NBUF - 1, steps)):
            hx[t] = start_x(t)

        for t in range(steps):
            c, b = divmod(t, batch)
            i = t % _NBUF
            if b == 0:
                hpos[c].wait()
                if c + 1 < n_chunks:
                    hpos[c + 1] = start_pos(c + 1)
            hx[t].wait()

            xb, pb = xbufs[i], pbufs[c % 2]

            @plsc.parallel_loop(0, chunk // _LANES, unroll=8)
            def _add(j):
                sl = pl.ds(j * _LANES, _LANES)
                plsc.addupdate(xb.at[sl], pb[sl])
            hout[t] = pltpu.async_copy(
                xb, out_hbm.at[pl.ds(x_off(t), chunk)], osems[i])

            p = t + _NBUF - 1
            if p < steps:
                q = p - _NBUF
                if q >= 0:
                    hout[q].wait()
                hx[p] = start_x(p)

        for t in range(max(0, steps - _NBUF), steps):
            hout[t].wait()

    return sc_k


def _sc_kernel(x, pos_table):
    batch, seq, dim = x.shape
    sc_k = _make_sc_kernel(batch, seq, dim)
    out = sc_k(x.reshape(-1), pos_table[:seq].reshape(-1))
    return out.reshape(batch, seq, dim)


def kernel(x, pos_table):
    return _tc_kernel(x, pos_table)
